# Initial kernel scaffold; baseline (speedup 1.0000x reference)
import functools
import math

import jax
import jax.numpy as jnp
from jax import lax
from jax.experimental import pallas as pl
from jax.experimental.pallas import tpu as pltpu
from jax.experimental.pallas import tpu_sc as plsc

D = 32
N = 4096 * 200
NC, NS = 2, 16
NW = NC * NS
ROWS_PER_W = N // NW
CHUNK = 1280
NCHUNK = ROWS_PER_W // CHUNK
SCALE = math.sqrt(D)

_mesh = plsc.VectorSubcoreMesh(
    core_axis_name="c", subcore_axis_name="s", num_cores=NC, num_subcores=NS
)


@functools.partial(
    pl.kernel,
    out_type=jax.ShapeDtypeStruct((N, D), jnp.float32),
    mesh=_mesh,
    scratch_types=[
        pltpu.VMEM((CHUNK,), jnp.int32),
        pltpu.VMEM((CHUNK, D), jnp.float32),
        pltpu.SemaphoreType.DMA,
    ],
)
def _embed_lookup(idx_hbm, table_hbm, out_hbm, idx_v, rows_v, sem):
    wid = lax.axis_index("s") * NC + lax.axis_index("c")
    base = wid * ROWS_PER_W

    def chunk_body(c, _):
        row0 = base + c * CHUNK
        pltpu.sync_copy(idx_hbm.at[pl.ds(row0, CHUNK)], idx_v)
        pltpu.async_copy(table_hbm.at[idx_v], rows_v, sem).wait()

        def scale_body(i, _):
            for j in range(2):
                sl = pl.ds(j * 16, 16)
                rows_v[i, sl] = rows_v[i, sl] * SCALE
            return 0

        lax.fori_loop(0, CHUNK, scale_body, 0, unroll=4)
        pltpu.sync_copy(rows_v, out_hbm.at[pl.ds(row0, CHUNK)])
        return 0

    lax.fori_loop(0, NCHUNK, chunk_body, 0)


def kernel(x, table):
    out = _embed_lookup(x.reshape(-1), table)
    return out.reshape(x.shape[0], x.shape[1], D)


# speedup vs baseline: 1.4093x; 1.4093x over previous
import functools
import math

import jax
import jax.numpy as jnp
from jax import lax
from jax.experimental import pallas as pl
from jax.experimental.pallas import tpu as pltpu
from jax.experimental.pallas import tpu_sc as plsc

D = 32
N = 4096 * 200
NC, NS = 2, 16
NW = NC * NS
ROWS_PER_W = N // NW
CHUNK = 1280
NCHUNK = ROWS_PER_W // CHUNK
SCALE = math.sqrt(D)

_mesh = plsc.VectorSubcoreMesh(
    core_axis_name="c", subcore_axis_name="s", num_cores=NC, num_subcores=NS
)


@functools.partial(
    pl.kernel,
    out_type=jax.ShapeDtypeStruct((N, D), jnp.float32),
    mesh=_mesh,
    scratch_types=[
        pltpu.VMEM((CHUNK,), jnp.int32),
        pltpu.VMEM((CHUNK, D), jnp.float32),
        pltpu.SemaphoreType.DMA,
    ],
    compiler_params=pltpu.CompilerParams(use_tc_tiling_on_sc=False),
)
def _embed_lookup(idx_hbm, table_hbm, out_hbm, idx_v, rows_v, sem):
    wid = lax.axis_index("s") * NC + lax.axis_index("c")
    base = wid * ROWS_PER_W

    def chunk_body(c, _):
        row0 = base + c * CHUNK
        pltpu.sync_copy(idx_hbm.at[pl.ds(row0, CHUNK)], idx_v)
        pltpu.async_copy(table_hbm.at[idx_v], rows_v, sem).wait()

        def scale_body(i, _):
            for j in range(2):
                sl = pl.ds(j * 16, 16)
                rows_v[i, sl] = rows_v[i, sl] * SCALE
            return 0

        lax.fori_loop(0, CHUNK, scale_body, 0, unroll=4)
        pltpu.sync_copy(rows_v, out_hbm.at[pl.ds(row0, CHUNK)])
        return 0

    lax.fori_loop(0, NCHUNK, chunk_body, 0)


def kernel(x, table):
    out = _embed_lookup(x.reshape(-1), table)
    return out.reshape(x.shape[0], x.shape[1], D)


# speedup vs baseline: 1.4745x; 1.0463x over previous
import functools
import math

import jax
import jax.numpy as jnp
from jax import lax
from jax.experimental import pallas as pl
from jax.experimental.pallas import tpu as pltpu
from jax.experimental.pallas import tpu_sc as plsc

D = 32
N = 4096 * 200
NC, NS = 2, 16
NW = NC * NS
ROWS_PER_W = N // NW
CHUNK = 1600
NCHUNK = ROWS_PER_W // CHUNK
SCALE = math.sqrt(D)

_mesh = plsc.VectorSubcoreMesh(
    core_axis_name="c", subcore_axis_name="s", num_cores=NC, num_subcores=NS
)


@functools.partial(
    pl.kernel,
    out_type=jax.ShapeDtypeStruct((N, D), jnp.float32),
    mesh=_mesh,
    scratch_types=[
        pltpu.VMEM((CHUNK,), jnp.int32),
        pltpu.VMEM((CHUNK,), jnp.int32),
        pltpu.VMEM((CHUNK, D), jnp.float32),
        pltpu.VMEM((CHUNK, D), jnp.float32),
        pltpu.SemaphoreType.DMA,
        pltpu.SemaphoreType.DMA,
        pltpu.SemaphoreType.DMA,
        pltpu.SemaphoreType.DMA,
    ],
    compiler_params=pltpu.CompilerParams(use_tc_tiling_on_sc=False),
)
def _embed_lookup(idx_hbm, table_hbm, out_hbm, idx0, idx1, rows0, rows1,
                  gsem0, gsem1, osem0, osem1):
    wid = lax.axis_index("s") * NC + lax.axis_index("c")
    base = wid * ROWS_PER_W

    def load_idx(c, idx_v):
        pltpu.sync_copy(idx_hbm.at[pl.ds(base + c * CHUNK, CHUNK)], idx_v)

    def start_gather(idx_v, rows_v, sem):
        return pltpu.async_copy(table_hbm.at[idx_v], rows_v, sem)

    def scale(rows_v):
        @plsc.parallel_loop(0, CHUNK, 1, unroll=8)
        def _(i):
            for j in range(2):
                sl = pl.ds(j * 16, 16)
                rows_v[i, sl] = rows_v[i, sl] * SCALE

    def start_out(c, rows_v, sem):
        return pltpu.async_copy(rows_v, out_hbm.at[pl.ds(base + c * CHUNK, CHUNK)], sem)

    def wait_gather(idx_v, rows_v, sem):
        pltpu.make_async_copy(table_hbm.at[idx_v], rows_v, sem).wait()

    def wait_out(c, rows_v, sem):
        pltpu.make_async_copy(rows_v, out_hbm.at[pl.ds(base + c * CHUNK, CHUNK)], sem).wait()

    load_idx(0, idx0)
    start_gather(idx0, rows0, gsem0)

    wait_gather(idx0, rows0, gsem0)
    load_idx(1, idx1)
    start_gather(idx1, rows1, gsem1)
    scale(rows0)
    start_out(0, rows0, osem0)

    def pair_body(s, _):
        c1 = 2 * s + 1
        c2 = 2 * s + 2
        wait_gather(idx1, rows1, gsem1)
        load_idx(c1 + 1, idx0)
        wait_out(c1 - 1, rows0, osem0)
        start_gather(idx0, rows0, gsem0)
        scale(rows1)
        start_out(c1, rows1, osem1)
        wait_gather(idx0, rows0, gsem0)
        load_idx(c2 + 1, idx1)
        wait_out(c2 - 1, rows1, osem1)
        start_gather(idx1, rows1, gsem1)
        scale(rows0)
        start_out(c2, rows0, osem0)
        return 0

    lax.fori_loop(0, (NCHUNK - 2) // 2, pair_body, 0)

    wait_gather(idx1, rows1, gsem1)
    scale(rows1)
    start_out(NCHUNK - 1, rows1, osem1)
    wait_out(NCHUNK - 2, rows0, osem0)
    wait_out(NCHUNK - 1, rows1, osem1)


def kernel(x, table):
    out = _embed_lookup(x.reshape(-1), table)
    return out.reshape(x.shape[0], x.shape[1], D)
